# baseline (speedup 1.0000x reference)
import functools

import jax
import jax.numpy as jnp
from jax import lax
from jax.experimental import pallas as pl
from jax.experimental.pallas import tpu as pltpu
from jax.experimental.pallas import tpu_sc as plsc

B = 4096
NUM = 13
CAT = 26
V = 100000
D = 64
F = NUM + CAT
L = 16


def _lane_bcast(v, k):
    idx = jnp.full((L, 1), k, jnp.int32)
    return lax.gather(
        v, idx,
        lax.GatherDimensionNumbers(offset_dims=(), collapsed_slice_dims=(0,),
                                   start_index_map=(0,)),
        (1,), mode=lax.GatherScatterMode.PROMISE_IN_BOUNDS)


def _body(NC, NW, xT_hbm, tab_hbm, nemb_hbm, out_hbm,
          xT_v, nemb_v, idx_v, gbuf_v, nbuf_v, sem0, sem1):
    sems = (sem0, sem1)
    bpw = B // NW
    wid = lax.axis_index("s") * NC + lax.axis_index("c")
    base = wid * bpw

    pltpu.sync_copy(xT_hbm.at[:, pl.ds(base, bpw)], xT_v)
    pltpu.sync_copy(nemb_hbm, nemb_v)

    def fill_idx(buf, j):
        def ib(i0, c):
            v = xT_v[NUM + j, pl.ds(i0 * L, L)]
            idx_v[buf, pl.ds(i0 * L, L)] = v + j * V
            return c
        lax.fori_loop(0, bpw // L, ib, 0)

    def fire(buf, j):
        fill_idx(buf, j)
        pltpu.async_copy(tab_hbm.at[idx_v.at[buf]], gbuf_v.at[buf], sems[buf])

    def wait_g(buf):
        pltpu.make_async_copy(
            tab_hbm.at[idx_v.at[buf]], gbuf_v.at[buf], sems[buf]).wait()

    def write_out(buf, j):
        pltpu.sync_copy(gbuf_v.at[buf],
                        out_hbm.at[pl.ds(base, bpw), NUM + j, :])

    fire(0, 0)

    def cat_body(t, c):
        j = 2 * t
        fire(1, j + 1)
        wait_g(0)
        write_out(0, j)

        @pl.when(j + 2 < CAT)
        def _():
            fire(0, j + 2)

        wait_g(1)
        write_out(1, j + 1)
        return c

    lax.fori_loop(0, CAT // 2, cat_body, 0)

    def num_body(j, c):
        e = [nemb_v[j, pl.ds(dc * L, L)] for dc in range(D // L)]

        def ib(i0, c2):
            xv = xT_v[j, pl.ds(i0 * L, L)].astype(jnp.float32)
            for k in range(L):
                bv = _lane_bcast(xv, k)
                for dc in range(D // L):
                    nbuf_v[i0 * L + k, pl.ds(dc * L, L)] = bv * e[dc]
            return c2
        lax.fori_loop(0, bpw // L, ib, 0)
        pltpu.sync_copy(nbuf_v, out_hbm.at[pl.ds(base, bpw), j, :])
        return c

    lax.fori_loop(0, NUM, num_body, 0)


def kernel(x, num_emb, cat_tables):
    info = plsc.get_sparse_core_info()
    NC, NS = info.num_cores, info.num_subcores
    NW = NC * NS
    bpw = B // NW

    xT = x.T
    tab = cat_tables.reshape(CAT * V, D)

    mesh = plsc.VectorSubcoreMesh(core_axis_name="c", subcore_axis_name="s")
    k = pl.kernel(
        functools.partial(_body, NC, NW),
        out_type=jax.ShapeDtypeStruct((B, F, D), jnp.float32),
        mesh=mesh,
        compiler_params=pltpu.CompilerParams(use_tc_tiling_on_sc=False),
        scratch_types=[
            pltpu.VMEM((F, bpw), jnp.int32),
            pltpu.VMEM((NUM, D), jnp.float32),
            pltpu.VMEM((2, bpw), jnp.int32),
            pltpu.VMEM((2, bpw, D), jnp.float32),
            pltpu.VMEM((bpw, D), jnp.float32),
            pltpu.SemaphoreType.DMA,
            pltpu.SemaphoreType.DMA,
        ],
    )
    return k(xT, tab, num_emb)


# speedup vs baseline: 4.1174x; 4.1174x over previous
import functools

import jax
import jax.numpy as jnp
from jax import lax
from jax.experimental import pallas as pl
from jax.experimental.pallas import tpu as pltpu
from jax.experimental.pallas import tpu_sc as plsc

B = 4096
NUM = 13
CAT = 26
V = 100000
D = 64
F = NUM + CAT
L = 16


def _lane_bcast(v, k):
    idx = jnp.full((L, 1), 1, jnp.int32) * k
    return lax.gather(
        v, idx,
        lax.GatherDimensionNumbers(offset_dims=(), collapsed_slice_dims=(0,),
                                   start_index_map=(0,)),
        (1,), mode=lax.GatherScatterMode.PROMISE_IN_BOUNDS)


def _body(NC, NW, xT_hbm, tab_hbm, nemb_hbm, out_hbm,
          slab_v, xint_v, obuf_v, nbuf_v, nemb_v,
          sem_s, sem_w0, sem_w1, sem_n):
    CPW = (CAT * D) // NW
    NPW = (NUM * D) // NW
    wid = lax.axis_index("s") * NC + lax.axis_index("c")
    c0 = wid * CPW
    n0 = wid * NPW

    pltpu.sync_copy(nemb_hbm, nemb_v)

    sems_w = (sem_w0, sem_w1)

    def out_row(f, d):
        return out_hbm.at[f, d, :]

    def wait_write(buf, f, d):
        pltpu.make_async_copy(obuf_v.at[buf], out_row(f, d),
                              sems_w[buf]).wait()

    def numerical_unit(n):
        j = n // D
        d = n % D
        pltpu.sync_copy(xT_hbm.at[j, :], xint_v)
        ev16 = nemb_v[j, pl.ds((d // L) * L, L)]
        ev = _lane_bcast(ev16, d % L)

        def cb(c, carry):
            xv = xint_v[pl.ds(c * L, L)].astype(jnp.float32)
            nbuf_v[pl.ds(c * L, L)] = xv * ev
            return carry
        lax.fori_loop(0, B // L, cb, 0)
        pltpu.async_copy(nbuf_v, out_row(j, d), sem_n)
        pltpu.make_async_copy(nbuf_v, out_row(j, d), sem_n).wait()

    def cat_unit(t, buf):
        p = c0 + t
        j = p // D
        d = p % D
        pltpu.async_copy(tab_hbm.at[j, d, :], slab_v, sem_s)

        @pl.when(t < NPW)
        def _():
            numerical_unit(n0 + t)

        pltpu.sync_copy(xT_hbm.at[NUM + j, :], xint_v)
        pltpu.make_async_copy(tab_hbm.at[j, d, :], slab_v, sem_s).wait()

        @pl.when(t >= 2)
        def _():
            pp = p - 2
            wait_write(buf, NUM + pp // D, pp % D)

        def gb(c, carry2):
            idxv = xint_v[pl.ds(c * L, L)]
            obuf_v[buf, pl.ds(c * L, L)] = plsc.load_gather(slab_v, [idxv])
            return carry2
        lax.fori_loop(0, B // L, gb, 0)
        pltpu.async_copy(obuf_v.at[buf], out_row(NUM + j, d), sems_w[buf])

    def pair_body(q, carry):
        cat_unit(2 * q, 0)
        cat_unit(2 * q + 1, 1)
        return carry

    lax.fori_loop(0, CPW // 2, pair_body, 0)

    for t in (CPW - 2, CPW - 1):
        p = c0 + t
        wait_write(t % 2, NUM + p // D, p % D)


def kernel(x, num_emb, cat_tables):
    info = plsc.get_sparse_core_info()
    NC, NS = info.num_cores, info.num_subcores
    NW = NC * NS

    xT = x.T
    tabT = jnp.transpose(cat_tables, (0, 2, 1))

    mesh = plsc.VectorSubcoreMesh(core_axis_name="c", subcore_axis_name="s")
    k = pl.kernel(
        functools.partial(_body, NC, NW),
        out_type=jax.ShapeDtypeStruct((F, D, B), jnp.float32),
        mesh=mesh,
        compiler_params=pltpu.CompilerParams(needs_layout_passes=False),
        scratch_types=[
            pltpu.VMEM((V,), jnp.float32),
            pltpu.VMEM((B,), jnp.int32),
            pltpu.VMEM((2, B), jnp.float32),
            pltpu.VMEM((B,), jnp.float32),
            pltpu.VMEM((NUM, D), jnp.float32),
            pltpu.SemaphoreType.DMA,
            pltpu.SemaphoreType.DMA,
            pltpu.SemaphoreType.DMA,
            pltpu.SemaphoreType.DMA,
        ],
    )
    out3 = k(xT, tabT, num_emb)
    return jnp.transpose(out3, (2, 0, 1))
